# Initial kernel scaffold; baseline (speedup 1.0000x reference)
import jax
import jax.numpy as jnp
from jax.experimental import pallas as pl


def kernel(full_state, edge_index, W_fe1, W_fe2, Wg0, Wb0, Wout0, Wg1, Wb1, Wout1):
    raise NotImplementedError("write your pallas kernel here")


# speedup vs baseline: 122.7194x; 122.7194x over previous
import functools

import jax
import jax.numpy as jnp
from jax import lax
from jax.experimental import pallas as pl
from jax.experimental.pallas import tpu as pltpu
from jax.experimental.pallas import tpu_sc as plsc

N = 100000
E = 1600000
N_FE = 32

NC = 2
NS = 16
U = E // 128
BN = 1000
NB = N // BN

_UH = U // NC
_QA, _RA = divmod(_UH, NS)
_QB, _RB = divmod(U, NS)

_ROWS_T = 6256
_NPAD = NS * _ROWS_T
_ZCH = 48
_TAIL = _ROWS_T - _ZCH * 128
_TAIL_L = N - 15 * _ROWS_T - _ZCH * 128


def _tile_range(s, q, r):
    nu = jnp.where(s < r, q + 1, q)
    ub = (q + 1) * jnp.minimum(s, r) + q * jnp.maximum(s - r, 0)
    return ub, nu


def _iota16():
    return lax.broadcasted_iota(jnp.int32, (16,), 0)


def _col(k):
    return jnp.full((16,), k, jnp.int32)


def _mesh():
    return plsc.VectorSubcoreMesh(core_axis_name="c", subcore_axis_name="s")


def _zero_stage(stage, width):
    if width == 16:
        def zb(i, carry):
            stage[i, :] = jnp.zeros((16,), jnp.float32)
            return carry
        lax.fori_loop(0, 128, zb, 0)
    else:
        ii = _iota16()
        z = jnp.zeros((16,), jnp.float32)
        for g in range(64):
            rr = g * 2 + ii // 8
            cc = ii % 8
            plsc.store_scatter(stage, [rr, cc], z)


def _zero_acc(stage, acc, s):
    row0 = s * _ROWS_T

    def zc(q, carry):
        pltpu.sync_copy(stage, acc.at[pl.ds(row0 + q * 128, 128)])
        return carry
    lax.fori_loop(0, _ZCH, zc, 0)
    pltpu.sync_copy(stage.at[pl.ds(0, _TAIL)],
                    acc.at[pl.ds(row0 + _ZCH * 128, _TAIL)])


def _drain_acc(acc, out_ref, s, base_out):
    row0 = s * _ROWS_T

    def dc(q, carry):
        pltpu.sync_copy(acc.at[pl.ds(row0 + q * 128, 128)],
                        out_ref.at[pl.ds(base_out + row0 + q * 128, 128)])
        return carry
    lax.fori_loop(0, _ZCH, dc, 0)

    tb = row0 + _ZCH * 128

    @pl.when(s < NS - 1)
    def _():
        pltpu.sync_copy(acc.at[pl.ds(tb, _TAIL)],
                        out_ref.at[pl.ds(base_out + tb, _TAIL)])

    @pl.when(s == NS - 1)
    def _():
        pltpu.sync_copy(acc.at[pl.ds(tb, _TAIL_L)],
                        out_ref.at[pl.ds(base_out + tb, _TAIL_L)])


def _p1a_body(src_ref, dst_ref, fsp_ref, out_ref,
              acc, sidx, didx, rows_s, rows_d, stage, sem1, sem2):
    c = lax.axis_index("c")
    s = lax.axis_index("s")
    _zero_stage(stage, 16)
    _zero_acc(stage, acc, s)
    plsc.subcore_barrier()

    ub, nu = _tile_range(s, _QA, _RA)
    ub = ub + c * _UH

    def unit(g, carry):
        eb = (ub + g) * 128
        pltpu.sync_copy(src_ref.at[pl.ds(eb, 128)], sidx)
        pltpu.sync_copy(dst_ref.at[pl.ds(eb, 128)], didx)
        cp1 = pltpu.async_copy(fsp_ref.at[sidx], rows_s, sem1)
        cp2 = pltpu.async_copy(fsp_ref.at[didx], rows_d, sem2)
        cp1.wait()
        cp2.wait()
        for gi in range(8):
            ri = _iota16() + gi * 16
            psx = plsc.load_gather(rows_s, [ri, _col(0)])
            psy = plsc.load_gather(rows_s, [ri, _col(1)])
            pdx = plsc.load_gather(rows_d, [ri, _col(0)])
            pdy = plsc.load_gather(rows_d, [ri, _col(1)])
            s0 = plsc.load_gather(rows_s, [ri, _col(2)])
            s1 = plsc.load_gather(rows_s, [ri, _col(3)])
            s2 = plsc.load_gather(rows_s, [ri, _col(4)])
            d0 = plsc.load_gather(rows_d, [ri, _col(2)])
            d1 = plsc.load_gather(rows_d, [ri, _col(3)])
            d2 = plsc.load_gather(rows_d, [ri, _col(4)])
            dx = pdx - psx
            dy = pdy - psy
            w = 1.0 / (dx * dx + dy * dy + 1e-6)
            du0 = d1 - s1
            du1 = d2 - s2
            plsc.store_scatter(stage, [ri, _col(0)], jnp.full((16,), 1.0, jnp.float32))
            plsc.store_scatter(stage, [ri, _col(1)], w * (s0 - d0))
            plsc.store_scatter(stage, [ri, _col(2)], w * (s1 - d1))
            plsc.store_scatter(stage, [ri, _col(3)], w * (s2 - d2))
            plsc.store_scatter(stage, [ri, _col(4)], dx * dx)
            plsc.store_scatter(stage, [ri, _col(5)], dx * dy)
            plsc.store_scatter(stage, [ri, _col(6)], dy * dy)
            plsc.store_scatter(stage, [ri, _col(7)], dx * du0)
            plsc.store_scatter(stage, [ri, _col(8)], dx * du1)
            plsc.store_scatter(stage, [ri, _col(9)], dy * du0)
            plsc.store_scatter(stage, [ri, _col(10)], dy * du1)
        pltpu.sync_copy(stage, acc.at[didx], add=True)
        return carry

    lax.fori_loop(0, nu, unit, 0)
    plsc.subcore_barrier()
    _drain_acc(acc, out_ref, s, c * N)


def _p1a(src, dst, fsp):
    f = functools.partial(
        pl.kernel,
        out_type=jax.ShapeDtypeStruct((2 * N, 16), jnp.float32),
        mesh=_mesh(),
        scratch_types=[
            pltpu.VMEM_SHARED((_NPAD, 16), jnp.float32),
            pltpu.VMEM((128,), jnp.int32),
            pltpu.VMEM((128,), jnp.int32),
            pltpu.VMEM((128, 8), jnp.float32),
            pltpu.VMEM((128, 8), jnp.float32),
            pltpu.VMEM((128, 16), jnp.float32),
            pltpu.SemaphoreType.DMA,
            pltpu.SemaphoreType.DMA,
        ],
        compiler_params=pltpu.CompilerParams(needs_layout_passes=False, use_tc_tiling_on_sc=False),
        name="p1a_geometry",
    )(_p1a_body)
    return f(src, dst, fsp)


def _p1b_body(src_ref, dst_ref, h2_ref, out_ref,
              acc, sidx, didx, rows, stage, sem1):
    c = lax.axis_index("c")
    s = lax.axis_index("s")
    _zero_stage(stage, 16)
    _zero_acc(stage, acc, s)
    plsc.subcore_barrier()

    ub, nu = _tile_range(s, _QB, _RB)
    coff = c * N

    def unit(g, carry):
        eb = (ub + g) * 128
        pltpu.sync_copy(src_ref.at[pl.ds(eb, 128)], sidx)
        pltpu.sync_copy(dst_ref.at[pl.ds(eb, 128)], didx)
        for gi in range(8):
            sl = pl.ds(gi * 16, 16)
            sidx[sl] = sidx[sl] + coff
        pltpu.async_copy(h2_ref.at[sidx], rows, sem1).wait()
        pltpu.sync_copy(rows, acc.at[didx], add=True)
        return carry

    lax.fori_loop(0, nu, unit, 0)
    plsc.subcore_barrier()
    _drain_acc(acc, out_ref, s, c * N)


def _p1b(src, dst, h2):
    f = functools.partial(
        pl.kernel,
        out_type=jax.ShapeDtypeStruct((2 * N, 16), jnp.float32),
        mesh=_mesh(),
        scratch_types=[
            pltpu.VMEM_SHARED((_NPAD, 16), jnp.float32),
            pltpu.VMEM((128,), jnp.int32),
            pltpu.VMEM((128,), jnp.int32),
            pltpu.VMEM((128, 16), jnp.float32),
            pltpu.VMEM((128, 16), jnp.float32),
            pltpu.SemaphoreType.DMA,
        ],
        compiler_params=pltpu.CompilerParams(needs_layout_passes=False, use_tc_tiling_on_sc=False),
        name="p1b_agg",
    )(_p1b_body)
    return f(src, dst, h2)


def _p2_body(src_ref, dst_ref, y_ref, out_ref,
             acc, sidx, didx, rows, sem1):
    c = lax.axis_index("c")
    s = lax.axis_index("s")
    _zero_stage(rows, 8)
    _zero_acc(rows, acc, s)
    plsc.subcore_barrier()

    ub, nu = _tile_range(s, _QA, _RA)
    ub = ub + c * _UH

    def unit(g, carry):
        eb = (ub + g) * 128
        pltpu.sync_copy(src_ref.at[pl.ds(eb, 128)], sidx)
        pltpu.sync_copy(dst_ref.at[pl.ds(eb, 128)], didx)
        pltpu.async_copy(y_ref.at[sidx], rows, sem1).wait()
        pltpu.sync_copy(rows, acc.at[didx], add=True)
        return carry

    lax.fori_loop(0, nu, unit, 0)
    plsc.subcore_barrier()
    _drain_acc(acc, out_ref, s, c * N)


def _p2(src, dst, y):
    f = functools.partial(
        pl.kernel,
        out_type=jax.ShapeDtypeStruct((2 * N, 8), jnp.float32),
        mesh=_mesh(),
        scratch_types=[
            pltpu.VMEM_SHARED((_NPAD, 8), jnp.float32),
            pltpu.VMEM((128,), jnp.int32),
            pltpu.VMEM((128,), jnp.int32),
            pltpu.VMEM((128, 8), jnp.float32),
            pltpu.SemaphoreType.DMA,
        ],
        compiler_params=pltpu.CompilerParams(needs_layout_passes=False, use_tc_tiling_on_sc=False),
        name="p2_mar",
    )(_p2_body)
    return f(src, dst, y)


def _ka_body(pos_ref, w_ref, out_ref):
    c = pl.program_id(0)
    h = jnp.maximum(
        jnp.dot(pos_ref[...], w_ref[...], preferred_element_type=jnp.float32),
        0.0)
    out_ref[...] = jnp.where(c == 0, h[:, :16], h[:, 16:])


def _kernel_a(pos, W_fe1):
    return pl.pallas_call(
        _ka_body,
        grid=(NC, NB),
        in_specs=[
            pl.BlockSpec((BN, 2), lambda c, i: (i, 0)),
            pl.BlockSpec((2, N_FE), lambda c, i: (0, 0)),
        ],
        out_specs=pl.BlockSpec((BN, 16), lambda c, i: (c * NB + i, 0)),
        out_shape=jax.ShapeDtypeStruct((2 * N, 16), jnp.float32),
    )(pos, W_fe1)


def _kb_body(h1_ref, h2_ref, g1_ref, g2_ref, a1_ref, a2_ref,
             wfe2_ref, wg0_ref, wb0_ref, wout0_ref, wg1_ref, wb1_ref, wout1_ref,
             y_ref):
    h = jnp.concatenate([h1_ref[...], h2_ref[...]], axis=1)
    geo = g1_ref[...] + g2_ref[...]
    deg = jnp.maximum(geo[:, 0:1], 1.0)
    agg = jnp.concatenate([a1_ref[...], a2_ref[...]], axis=1) / deg
    wfe2 = wfe2_ref[...]
    learned = jnp.maximum(
        jnp.dot(h, wfe2[:N_FE], preferred_element_type=jnp.float32)
        + jnp.dot(agg, wfe2[N_FE:], preferred_element_type=jnp.float32), 0.0)

    lap0 = geo[:, 1:2] / deg
    lap1 = geo[:, 2:3] / deg
    lap2 = geo[:, 3:4] / deg
    a00 = geo[:, 4:5] + 1e-4
    a01 = geo[:, 5:6]
    a11 = geo[:, 6:7] + 1e-4
    b00 = geo[:, 7:8]
    b01 = geo[:, 8:9]
    b10 = geo[:, 9:10]
    b11 = geo[:, 10:11]
    det = a00 * a11 - a01 * a01
    g00 = (a11 * b00 - a01 * b10) / det
    g01 = (a11 * b01 - a01 * b11) / det
    g10 = (a00 * b10 - a01 * b00) / det
    g11 = (a00 * b11 - a01 * b01) / det
    exx = g00
    eyy = g11
    exy = 0.5 * (g01 + g10)
    vm = jnp.sqrt(exx * exx - exx * eyy + eyy * eyy + 3.0 * exy * exy + 1e-12)
    vol = exx + eyy

    mod0 = learned * (1.0 + lap0 * wg0_ref[...]) + lap0 * wb0_ref[...]
    y0 = jnp.dot(mod0, wout0_ref[...], preferred_element_type=jnp.float32)
    mask1 = jnp.concatenate([exx, eyy, exy, vm, vol, lap1, lap2], axis=1)
    mod1 = (learned * (1.0 + jnp.dot(mask1, wg1_ref[...],
                                     preferred_element_type=jnp.float32))
            + jnp.dot(mask1, wb1_ref[...], preferred_element_type=jnp.float32))
    y12 = jnp.dot(mod1, wout1_ref[...], preferred_element_type=jnp.float32)
    y_ref[...] = jnp.concatenate(
        [y0, y12, jnp.zeros((y0.shape[0], 5), jnp.float32)], axis=1)


def _kernel_b(h2, geo, agg, W_fe2, Wg0, Wb0, Wout0, Wg1, Wb1, Wout1):
    full = lambda shape: pl.BlockSpec(shape, lambda i: (0, 0))
    return pl.pallas_call(
        _kb_body,
        grid=(NB,),
        in_specs=[
            pl.BlockSpec((BN, 16), lambda i: (i, 0)),
            pl.BlockSpec((BN, 16), lambda i: (NB + i, 0)),
            pl.BlockSpec((BN, 16), lambda i: (i, 0)),
            pl.BlockSpec((BN, 16), lambda i: (NB + i, 0)),
            pl.BlockSpec((BN, 16), lambda i: (i, 0)),
            pl.BlockSpec((BN, 16), lambda i: (NB + i, 0)),
            full((2 * N_FE, N_FE)),
            full((1, N_FE)),
            full((1, N_FE)),
            full((N_FE, 1)),
            full((7, N_FE)),
            full((7, N_FE)),
            full((N_FE, 2)),
        ],
        out_specs=pl.BlockSpec((BN, 8), lambda i: (i, 0)),
        out_shape=jax.ShapeDtypeStruct((N, 8), jnp.float32),
    )(h2, h2, geo, geo, agg, agg, W_fe2, Wg0, Wb0, Wout0, Wg1, Wb1, Wout1)


def _kc_body(ya_ref, yb_ref, g1_ref, g2_ref, out_ref):
    deg = jnp.maximum(g1_ref[:, 0:1] + g2_ref[:, 0:1], 1.0)
    out_ref[...] = (ya_ref[:, :3] + yb_ref[:, :3]) / deg


def _kernel_c(acc_y, geo):
    return pl.pallas_call(
        _kc_body,
        grid=(NB,),
        in_specs=[
            pl.BlockSpec((BN, 8), lambda i: (i, 0)),
            pl.BlockSpec((BN, 8), lambda i: (NB + i, 0)),
            pl.BlockSpec((BN, 16), lambda i: (i, 0)),
            pl.BlockSpec((BN, 16), lambda i: (NB + i, 0)),
        ],
        out_specs=pl.BlockSpec((BN, 3), lambda i: (i, 0)),
        out_shape=jax.ShapeDtypeStruct((N, 3), jnp.float32),
    )(acc_y, acc_y, geo, geo)


def kernel(full_state, edge_index, W_fe1, W_fe2, Wg0, Wb0, Wout0, Wg1, Wb1, Wout1):
    src = edge_index[0]
    dst = edge_index[1]
    fsp = jnp.concatenate(
        [full_state, jnp.zeros((N, 3), jnp.float32)], axis=1)
    h2 = _kernel_a(full_state[:, :2], W_fe1)
    geo = _p1a(src, dst, fsp)
    agg = _p1b(src, dst, h2)
    y = _kernel_b(h2, geo, agg, W_fe2, Wg0, Wb0, Wout0, Wg1, Wb1, Wout1)
    acc_y = _p2(src, dst, y)
    return _kernel_c(acc_y, geo)


# speedup vs baseline: 259.8212x; 2.1172x over previous
import functools

import jax
import jax.numpy as jnp
from jax import lax
from jax.experimental import pallas as pl
from jax.experimental.pallas import tpu as pltpu
from jax.experimental.pallas import tpu_sc as plsc

N = 100000
E = 1600000
N_FE = 32

NC = 2
NS = 16
U = E // 128
BN = 1000
NB = N // BN

_UH = U // NC
_QA, _RA = divmod(_UH, NS)
_QB, _RB = divmod(U, NS)

_ROWS_T = 6256
_NPAD = NS * _ROWS_T
_ZCH = 48
_TAIL = _ROWS_T - _ZCH * 128
_TAIL_L = N - 15 * _ROWS_T - _ZCH * 128


def _tile_range(s, q, r):
    nu = jnp.where(s < r, q + 1, q)
    ub = (q + 1) * jnp.minimum(s, r) + q * jnp.maximum(s - r, 0)
    return ub, nu


def _iota16():
    return lax.broadcasted_iota(jnp.int32, (16,), 0)


def _col(k):
    return jnp.full((16,), k, jnp.int32)


def _mesh():
    return plsc.VectorSubcoreMesh(core_axis_name="c", subcore_axis_name="s")


def _zero_stage(stage, width):
    if width == 16:
        def zb(i, carry):
            stage[i, :] = jnp.zeros((16,), jnp.float32)
            return carry
        lax.fori_loop(0, 128, zb, 0)
    else:
        ii = _iota16()
        z = jnp.zeros((16,), jnp.float32)
        for g in range(64):
            rr = g * 2 + ii // 8
            cc = ii % 8
            plsc.store_scatter(stage, [rr, cc], z)


def _zero_acc(stage, acc, s):
    row0 = s * _ROWS_T

    def zc(q, carry):
        pltpu.sync_copy(stage, acc.at[pl.ds(row0 + q * 128, 128)])
        return carry
    lax.fori_loop(0, _ZCH, zc, 0)
    pltpu.sync_copy(stage.at[pl.ds(0, _TAIL)],
                    acc.at[pl.ds(row0 + _ZCH * 128, _TAIL)])


def _drain_acc(acc, out_ref, s, base_out):
    row0 = s * _ROWS_T

    def dc(q, carry):
        pltpu.sync_copy(acc.at[pl.ds(row0 + q * 128, 128)],
                        out_ref.at[pl.ds(base_out + row0 + q * 128, 128)])
        return carry
    lax.fori_loop(0, _ZCH, dc, 0)

    tb = row0 + _ZCH * 128

    @pl.when(s < NS - 1)
    def _():
        pltpu.sync_copy(acc.at[pl.ds(tb, _TAIL)],
                        out_ref.at[pl.ds(base_out + tb, _TAIL)])

    @pl.when(s == NS - 1)
    def _():
        pltpu.sync_copy(acc.at[pl.ds(tb, _TAIL_L)],
                        out_ref.at[pl.ds(base_out + tb, _TAIL_L)])


def _sw_pipeline(nu, nmax, issue_idx, start_gather, consume):
    @pl.when(0 < nu)
    def _():
        issue_idx(0, 0)

    @pl.when(1 < nu)
    def _():
        issue_idx(1, 1)

    def step(i, carry):
        for j in range(3):
            u = i * 3 + j
            k = j
            kp = (j + 2) % 3

            @pl.when(u < nu)
            def _():
                start_gather(u, k)

            @pl.when((u >= 1) & (u - 1 < nu))
            def _():
                consume(u - 1, kp)

            @pl.when(u + 2 < nu)
            def _():
                issue_idx(u + 2, kp)
        return carry
    lax.fori_loop(0, nmax // 3 + 1, step, 0)


def _p1a_body(src_ref, dst_ref, fsp_ref, out_ref, *scr):
    acc = scr[0]
    sidx = scr[1:4]
    didx = scr[4:7]
    rows_s = scr[7:10]
    rows_d = scr[10:13]
    stage = scr[13]
    si_sem = scr[14:17]
    di_sem = scr[17:20]
    gs_sem = scr[20:23]
    gd_sem = scr[23:26]
    c = lax.axis_index("c")
    s = lax.axis_index("s")
    _zero_stage(stage, 16)
    _zero_acc(stage, acc, s)
    plsc.subcore_barrier()

    ub, nu = _tile_range(s, _QA, _RA)
    ub = ub + c * _UH

    def issue_idx(u, k):
        eb = (ub + u) * 128
        pltpu.async_copy(src_ref.at[pl.ds(eb, 128)], sidx[k], si_sem[k])
        pltpu.async_copy(dst_ref.at[pl.ds(eb, 128)], didx[k], di_sem[k])

    def start_gather(u, k):
        eb = (ub + u) * 128
        pltpu.make_async_copy(src_ref.at[pl.ds(eb, 128)], sidx[k],
                              si_sem[k]).wait()
        pltpu.make_async_copy(dst_ref.at[pl.ds(eb, 128)], didx[k],
                              di_sem[k]).wait()
        pltpu.async_copy(fsp_ref.at[sidx[k]], rows_s[k], gs_sem[k])
        pltpu.async_copy(fsp_ref.at[didx[k]], rows_d[k], gd_sem[k])

    def consume(u, k):
        pltpu.make_async_copy(fsp_ref.at[sidx[k]], rows_s[k],
                              gs_sem[k]).wait()
        pltpu.make_async_copy(fsp_ref.at[didx[k]], rows_d[k],
                              gd_sem[k]).wait()
        rs = rows_s[k]
        rd = rows_d[k]
        for gi in range(8):
            ri = _iota16() + gi * 16
            psx = plsc.load_gather(rs, [ri, _col(0)])
            psy = plsc.load_gather(rs, [ri, _col(1)])
            pdx = plsc.load_gather(rd, [ri, _col(0)])
            pdy = plsc.load_gather(rd, [ri, _col(1)])
            s0 = plsc.load_gather(rs, [ri, _col(2)])
            s1 = plsc.load_gather(rs, [ri, _col(3)])
            s2 = plsc.load_gather(rs, [ri, _col(4)])
            d0 = plsc.load_gather(rd, [ri, _col(2)])
            d1 = plsc.load_gather(rd, [ri, _col(3)])
            d2 = plsc.load_gather(rd, [ri, _col(4)])
            dx = pdx - psx
            dy = pdy - psy
            w = 1.0 / (dx * dx + dy * dy + 1e-6)
            du0 = d1 - s1
            du1 = d2 - s2
            plsc.store_scatter(stage, [ri, _col(0)], jnp.full((16,), 1.0, jnp.float32))
            plsc.store_scatter(stage, [ri, _col(1)], w * (s0 - d0))
            plsc.store_scatter(stage, [ri, _col(2)], w * (s1 - d1))
            plsc.store_scatter(stage, [ri, _col(3)], w * (s2 - d2))
            plsc.store_scatter(stage, [ri, _col(4)], dx * dx)
            plsc.store_scatter(stage, [ri, _col(5)], dx * dy)
            plsc.store_scatter(stage, [ri, _col(6)], dy * dy)
            plsc.store_scatter(stage, [ri, _col(7)], dx * du0)
            plsc.store_scatter(stage, [ri, _col(8)], dx * du1)
            plsc.store_scatter(stage, [ri, _col(9)], dy * du0)
            plsc.store_scatter(stage, [ri, _col(10)], dy * du1)
        pltpu.sync_copy(stage, acc.at[didx[k]], add=True)

    _sw_pipeline(nu, _QA + 1, issue_idx, start_gather, consume)
    plsc.subcore_barrier()
    _drain_acc(acc, out_ref, s, c * N)


def _p1a(src, dst, fsp):
    f = functools.partial(
        pl.kernel,
        out_type=jax.ShapeDtypeStruct((2 * N, 16), jnp.float32),
        mesh=_mesh(),
        scratch_types=(
            [pltpu.VMEM_SHARED((_NPAD, 16), jnp.float32)]
            + [pltpu.VMEM((128,), jnp.int32)] * 6
            + [pltpu.VMEM((128, 8), jnp.float32)] * 6
            + [pltpu.VMEM((128, 16), jnp.float32)]
            + [pltpu.SemaphoreType.DMA] * 12
        ),
        compiler_params=pltpu.CompilerParams(needs_layout_passes=False, use_tc_tiling_on_sc=False),
        name="p1a_geometry",
    )(_p1a_body)
    return f(src, dst, fsp)


def _p1b_body(src_ref, dst_ref, h2_ref, out_ref, *scr):
    acc = scr[0]
    sidx = scr[1:4]
    didx = scr[4:7]
    rows = scr[7:10]
    stage = scr[10]
    si_sem = scr[11:14]
    di_sem = scr[14:17]
    g_sem = scr[17:20]
    c = lax.axis_index("c")
    s = lax.axis_index("s")
    _zero_stage(stage, 16)
    _zero_acc(stage, acc, s)
    plsc.subcore_barrier()

    ub, nu = _tile_range(s, _QB, _RB)
    coff = c * N

    def issue_idx(u, k):
        eb = (ub + u) * 128
        pltpu.async_copy(src_ref.at[pl.ds(eb, 128)], sidx[k], si_sem[k])
        pltpu.async_copy(dst_ref.at[pl.ds(eb, 128)], didx[k], di_sem[k])

    def start_gather(u, k):
        eb = (ub + u) * 128
        pltpu.make_async_copy(src_ref.at[pl.ds(eb, 128)], sidx[k],
                              si_sem[k]).wait()
        pltpu.make_async_copy(dst_ref.at[pl.ds(eb, 128)], didx[k],
                              di_sem[k]).wait()
        sx = sidx[k]
        for gi in range(8):
            sl = pl.ds(gi * 16, 16)
            sx[sl] = sx[sl] + coff
        pltpu.async_copy(h2_ref.at[sx], rows[k], g_sem[k])

    def consume(u, k):
        pltpu.make_async_copy(h2_ref.at[sidx[k]], rows[k], g_sem[k]).wait()
        pltpu.sync_copy(rows[k], acc.at[didx[k]], add=True)

    _sw_pipeline(nu, _QB + 1, issue_idx, start_gather, consume)
    plsc.subcore_barrier()
    _drain_acc(acc, out_ref, s, c * N)


def _p1b(src, dst, h2):
    f = functools.partial(
        pl.kernel,
        out_type=jax.ShapeDtypeStruct((2 * N, 16), jnp.float32),
        mesh=_mesh(),
        scratch_types=(
            [pltpu.VMEM_SHARED((_NPAD, 16), jnp.float32)]
            + [pltpu.VMEM((128,), jnp.int32)] * 6
            + [pltpu.VMEM((128, 16), jnp.float32)] * 3
            + [pltpu.VMEM((128, 16), jnp.float32)]
            + [pltpu.SemaphoreType.DMA] * 9
        ),
        compiler_params=pltpu.CompilerParams(needs_layout_passes=False, use_tc_tiling_on_sc=False),
        name="p1b_agg",
    )(_p1b_body)
    return f(src, dst, h2)


def _p2_body(src_ref, dst_ref, y_ref, out_ref, *scr):
    acc = scr[0]
    sidx = scr[1:4]
    didx = scr[4:7]
    rows = scr[7:10]
    si_sem = scr[10:13]
    di_sem = scr[13:16]
    g_sem = scr[16:19]
    c = lax.axis_index("c")
    s = lax.axis_index("s")
    _zero_stage(rows[0], 8)
    _zero_acc(rows[0], acc, s)
    plsc.subcore_barrier()

    ub, nu = _tile_range(s, _QA, _RA)
    ub = ub + c * _UH

    def issue_idx(u, k):
        eb = (ub + u) * 128
        pltpu.async_copy(src_ref.at[pl.ds(eb, 128)], sidx[k], si_sem[k])
        pltpu.async_copy(dst_ref.at[pl.ds(eb, 128)], didx[k], di_sem[k])

    def start_gather(u, k):
        eb = (ub + u) * 128
        pltpu.make_async_copy(src_ref.at[pl.ds(eb, 128)], sidx[k],
                              si_sem[k]).wait()
        pltpu.make_async_copy(dst_ref.at[pl.ds(eb, 128)], didx[k],
                              di_sem[k]).wait()
        pltpu.async_copy(y_ref.at[sidx[k]], rows[k], g_sem[k])

    def consume(u, k):
        pltpu.make_async_copy(y_ref.at[sidx[k]], rows[k], g_sem[k]).wait()
        pltpu.sync_copy(rows[k], acc.at[didx[k]], add=True)

    _sw_pipeline(nu, _QA + 1, issue_idx, start_gather, consume)
    plsc.subcore_barrier()
    _drain_acc(acc, out_ref, s, c * N)


def _p2(src, dst, y):
    f = functools.partial(
        pl.kernel,
        out_type=jax.ShapeDtypeStruct((2 * N, 8), jnp.float32),
        mesh=_mesh(),
        scratch_types=(
            [pltpu.VMEM_SHARED((_NPAD, 8), jnp.float32)]
            + [pltpu.VMEM((128,), jnp.int32)] * 6
            + [pltpu.VMEM((128, 8), jnp.float32)] * 3
            + [pltpu.SemaphoreType.DMA] * 9
        ),
        compiler_params=pltpu.CompilerParams(needs_layout_passes=False, use_tc_tiling_on_sc=False),
        name="p2_mar",
    )(_p2_body)
    return f(src, dst, y)


def _ka_body(pos_ref, w_ref, out_ref):
    c = pl.program_id(0)
    h = jnp.maximum(
        jnp.dot(pos_ref[...], w_ref[...], preferred_element_type=jnp.float32),
        0.0)
    out_ref[...] = jnp.where(c == 0, h[:, :16], h[:, 16:])


def _kernel_a(pos, W_fe1):
    return pl.pallas_call(
        _ka_body,
        grid=(NC, NB),
        in_specs=[
            pl.BlockSpec((BN, 2), lambda c, i: (i, 0)),
            pl.BlockSpec((2, N_FE), lambda c, i: (0, 0)),
        ],
        out_specs=pl.BlockSpec((BN, 16), lambda c, i: (c * NB + i, 0)),
        out_shape=jax.ShapeDtypeStruct((2 * N, 16), jnp.float32),
    )(pos, W_fe1)


def _kb_body(h1_ref, h2_ref, g1_ref, g2_ref, a1_ref, a2_ref,
             wfe2_ref, wg0_ref, wb0_ref, wout0_ref, wg1_ref, wb1_ref, wout1_ref,
             y_ref):
    h = jnp.concatenate([h1_ref[...], h2_ref[...]], axis=1)
    geo = g1_ref[...] + g2_ref[...]
    deg = jnp.maximum(geo[:, 0:1], 1.0)
    agg = jnp.concatenate([a1_ref[...], a2_ref[...]], axis=1) / deg
    wfe2 = wfe2_ref[...]
    learned = jnp.maximum(
        jnp.dot(h, wfe2[:N_FE], preferred_element_type=jnp.float32)
        + jnp.dot(agg, wfe2[N_FE:], preferred_element_type=jnp.float32), 0.0)

    lap0 = geo[:, 1:2] / deg
    lap1 = geo[:, 2:3] / deg
    lap2 = geo[:, 3:4] / deg
    a00 = geo[:, 4:5] + 1e-4
    a01 = geo[:, 5:6]
    a11 = geo[:, 6:7] + 1e-4
    b00 = geo[:, 7:8]
    b01 = geo[:, 8:9]
    b10 = geo[:, 9:10]
    b11 = geo[:, 10:11]
    det = a00 * a11 - a01 * a01
    g00 = (a11 * b00 - a01 * b10) / det
    g01 = (a11 * b01 - a01 * b11) / det
    g10 = (a00 * b10 - a01 * b00) / det
    g11 = (a00 * b11 - a01 * b01) / det
    exx = g00
    eyy = g11
    exy = 0.5 * (g01 + g10)
    vm = jnp.sqrt(exx * exx - exx * eyy + eyy * eyy + 3.0 * exy * exy + 1e-12)
    vol = exx + eyy

    mod0 = learned * (1.0 + lap0 * wg0_ref[...]) + lap0 * wb0_ref[...]
    y0 = jnp.dot(mod0, wout0_ref[...], preferred_element_type=jnp.float32)
    mask1 = jnp.concatenate([exx, eyy, exy, vm, vol, lap1, lap2], axis=1)
    mod1 = (learned * (1.0 + jnp.dot(mask1, wg1_ref[...],
                                     preferred_element_type=jnp.float32))
            + jnp.dot(mask1, wb1_ref[...], preferred_element_type=jnp.float32))
    y12 = jnp.dot(mod1, wout1_ref[...], preferred_element_type=jnp.float32)
    y_ref[...] = jnp.concatenate(
        [y0, y12, jnp.zeros((y0.shape[0], 5), jnp.float32)], axis=1)


def _kernel_b(h2, geo, agg, W_fe2, Wg0, Wb0, Wout0, Wg1, Wb1, Wout1):
    full = lambda shape: pl.BlockSpec(shape, lambda i: (0, 0))
    return pl.pallas_call(
        _kb_body,
        grid=(NB,),
        in_specs=[
            pl.BlockSpec((BN, 16), lambda i: (i, 0)),
            pl.BlockSpec((BN, 16), lambda i: (NB + i, 0)),
            pl.BlockSpec((BN, 16), lambda i: (i, 0)),
            pl.BlockSpec((BN, 16), lambda i: (NB + i, 0)),
            pl.BlockSpec((BN, 16), lambda i: (i, 0)),
            pl.BlockSpec((BN, 16), lambda i: (NB + i, 0)),
            full((2 * N_FE, N_FE)),
            full((1, N_FE)),
            full((1, N_FE)),
            full((N_FE, 1)),
            full((7, N_FE)),
            full((7, N_FE)),
            full((N_FE, 2)),
        ],
        out_specs=pl.BlockSpec((BN, 8), lambda i: (i, 0)),
        out_shape=jax.ShapeDtypeStruct((N, 8), jnp.float32),
    )(h2, h2, geo, geo, agg, agg, W_fe2, Wg0, Wb0, Wout0, Wg1, Wb1, Wout1)


def _kc_body(ya_ref, yb_ref, g1_ref, g2_ref, out_ref):
    deg = jnp.maximum(g1_ref[:, 0:1] + g2_ref[:, 0:1], 1.0)
    out_ref[...] = (ya_ref[:, :3] + yb_ref[:, :3]) / deg


def _kernel_c(acc_y, geo):
    return pl.pallas_call(
        _kc_body,
        grid=(NB,),
        in_specs=[
            pl.BlockSpec((BN, 8), lambda i: (i, 0)),
            pl.BlockSpec((BN, 8), lambda i: (NB + i, 0)),
            pl.BlockSpec((BN, 16), lambda i: (i, 0)),
            pl.BlockSpec((BN, 16), lambda i: (NB + i, 0)),
        ],
        out_specs=pl.BlockSpec((BN, 3), lambda i: (i, 0)),
        out_shape=jax.ShapeDtypeStruct((N, 3), jnp.float32),
    )(acc_y, acc_y, geo, geo)


def kernel(full_state, edge_index, W_fe1, W_fe2, Wg0, Wb0, Wout0, Wg1, Wb1, Wout1):
    src = edge_index[0]
    dst = edge_index[1]
    fsp = jnp.concatenate(
        [full_state, jnp.zeros((N, 3), jnp.float32)], axis=1)
    h2 = _kernel_a(full_state[:, :2], W_fe1)
    geo = _p1a(src, dst, fsp)
    agg = _p1b(src, dst, h2)
    y = _kernel_b(h2, geo, agg, W_fe2, Wg0, Wb0, Wout0, Wg1, Wb1, Wout1)
    acc_y = _p2(src, dst, y)
    return _kernel_c(acc_y, geo)


# speedup vs baseline: 265.4446x; 1.0216x over previous
import functools

import jax
import jax.numpy as jnp
from jax import lax
from jax.experimental import pallas as pl
from jax.experimental.pallas import tpu as pltpu
from jax.experimental.pallas import tpu_sc as plsc

N = 100000
E = 1600000
N_FE = 32

NC = 2
NS = 16
U = E // 128
BN = 2000
NB = N // BN

_UH = U // NC
_QA, _RA = divmod(_UH, NS)
_QB, _RB = divmod(U, NS)

_ROWS_T = 6256
_NPAD = NS * _ROWS_T
_ZCH = 48
_TAIL = _ROWS_T - _ZCH * 128
_TAIL_L = N - 15 * _ROWS_T - _ZCH * 128


def _tile_range(s, q, r):
    nu = jnp.where(s < r, q + 1, q)
    ub = (q + 1) * jnp.minimum(s, r) + q * jnp.maximum(s - r, 0)
    return ub, nu


def _iota16():
    return lax.broadcasted_iota(jnp.int32, (16,), 0)


def _col(k):
    return jnp.full((16,), k, jnp.int32)


def _mesh():
    return plsc.VectorSubcoreMesh(core_axis_name="c", subcore_axis_name="s")


def _zero_stage(stage, width):
    if width == 16:
        def zb(i, carry):
            stage[i, :] = jnp.zeros((16,), jnp.float32)
            return carry
        lax.fori_loop(0, 128, zb, 0)
    else:
        ii = _iota16()
        z = jnp.zeros((16,), jnp.float32)
        for g in range(64):
            rr = g * 2 + ii // 8
            cc = ii % 8
            plsc.store_scatter(stage, [rr, cc], z)


def _zero_acc(stage, acc, s):
    row0 = s * _ROWS_T

    def zc(q, carry):
        pltpu.sync_copy(stage, acc.at[pl.ds(row0 + q * 128, 128)])
        return carry
    lax.fori_loop(0, _ZCH, zc, 0)
    pltpu.sync_copy(stage.at[pl.ds(0, _TAIL)],
                    acc.at[pl.ds(row0 + _ZCH * 128, _TAIL)])


def _drain_acc(acc, out_ref, s, base_out):
    row0 = s * _ROWS_T

    def dc(q, carry):
        pltpu.sync_copy(acc.at[pl.ds(row0 + q * 128, 128)],
                        out_ref.at[pl.ds(base_out + row0 + q * 128, 128)])
        return carry
    lax.fori_loop(0, _ZCH, dc, 0)

    tb = row0 + _ZCH * 128

    @pl.when(s < NS - 1)
    def _():
        pltpu.sync_copy(acc.at[pl.ds(tb, _TAIL)],
                        out_ref.at[pl.ds(base_out + tb, _TAIL)])

    @pl.when(s == NS - 1)
    def _():
        pltpu.sync_copy(acc.at[pl.ds(tb, _TAIL_L)],
                        out_ref.at[pl.ds(base_out + tb, _TAIL_L)])


def _sw_pipeline(nu, nmax, issue_idx, start_gather, consume):
    @pl.when(0 < nu)
    def _():
        issue_idx(0, 0)

    @pl.when(1 < nu)
    def _():
        issue_idx(1, 1)

    def step(i, carry):
        for j in range(3):
            u = i * 3 + j
            k = j
            kp = (j + 2) % 3

            @pl.when(u < nu)
            def _():
                start_gather(u, k)

            @pl.when((u >= 1) & (u - 1 < nu))
            def _():
                consume(u - 1, kp)

            @pl.when(u + 2 < nu)
            def _():
                issue_idx(u + 2, kp)
        return carry
    lax.fori_loop(0, nmax // 3 + 1, step, 0)


def _p1a_body(src_ref, dst_ref, fsp_ref, out_ref, *scr):
    acc = scr[0]
    sidx = scr[1:4]
    didx = scr[4:7]
    rows_s = scr[7:10]
    rows_d = scr[10:13]
    stage = scr[13]
    si_sem = scr[14:17]
    di_sem = scr[17:20]
    gs_sem = scr[20:23]
    gd_sem = scr[23:26]
    c = lax.axis_index("c")
    s = lax.axis_index("s")
    _zero_stage(stage, 16)
    _zero_acc(stage, acc, s)
    plsc.subcore_barrier()

    ub, nu = _tile_range(s, _QA, _RA)
    ub = ub + c * _UH

    def issue_idx(u, k):
        eb = (ub + u) * 128
        pltpu.async_copy(src_ref.at[pl.ds(eb, 128)], sidx[k], si_sem[k])
        pltpu.async_copy(dst_ref.at[pl.ds(eb, 128)], didx[k], di_sem[k])

    def start_gather(u, k):
        eb = (ub + u) * 128
        pltpu.make_async_copy(src_ref.at[pl.ds(eb, 128)], sidx[k],
                              si_sem[k]).wait()
        pltpu.make_async_copy(dst_ref.at[pl.ds(eb, 128)], didx[k],
                              di_sem[k]).wait()
        pltpu.async_copy(fsp_ref.at[sidx[k]], rows_s[k], gs_sem[k])
        pltpu.async_copy(fsp_ref.at[didx[k]], rows_d[k], gd_sem[k])

    def consume(u, k):
        pltpu.make_async_copy(fsp_ref.at[sidx[k]], rows_s[k],
                              gs_sem[k]).wait()
        pltpu.make_async_copy(fsp_ref.at[didx[k]], rows_d[k],
                              gd_sem[k]).wait()
        rs = rows_s[k]
        rd = rows_d[k]
        for gi in range(8):
            ri = _iota16() + gi * 16
            psx = plsc.load_gather(rs, [ri, _col(0)])
            psy = plsc.load_gather(rs, [ri, _col(1)])
            pdx = plsc.load_gather(rd, [ri, _col(0)])
            pdy = plsc.load_gather(rd, [ri, _col(1)])
            s0 = plsc.load_gather(rs, [ri, _col(2)])
            s1 = plsc.load_gather(rs, [ri, _col(3)])
            s2 = plsc.load_gather(rs, [ri, _col(4)])
            d0 = plsc.load_gather(rd, [ri, _col(2)])
            d1 = plsc.load_gather(rd, [ri, _col(3)])
            d2 = plsc.load_gather(rd, [ri, _col(4)])
            dx = pdx - psx
            dy = pdy - psy
            w = 1.0 / (dx * dx + dy * dy + 1e-6)
            du0 = d1 - s1
            du1 = d2 - s2
            plsc.store_scatter(stage, [ri, _col(0)], jnp.full((16,), 1.0, jnp.float32))
            plsc.store_scatter(stage, [ri, _col(1)], w * (s0 - d0))
            plsc.store_scatter(stage, [ri, _col(2)], w * (s1 - d1))
            plsc.store_scatter(stage, [ri, _col(3)], w * (s2 - d2))
            plsc.store_scatter(stage, [ri, _col(4)], dx * dx)
            plsc.store_scatter(stage, [ri, _col(5)], dx * dy)
            plsc.store_scatter(stage, [ri, _col(6)], dy * dy)
            plsc.store_scatter(stage, [ri, _col(7)], dx * du0)
            plsc.store_scatter(stage, [ri, _col(8)], dx * du1)
            plsc.store_scatter(stage, [ri, _col(9)], dy * du0)
            plsc.store_scatter(stage, [ri, _col(10)], dy * du1)
        pltpu.sync_copy(stage, acc.at[didx[k]], add=True)

    _sw_pipeline(nu, _QA + 1, issue_idx, start_gather, consume)
    plsc.subcore_barrier()
    _drain_acc(acc, out_ref, s, c * N)


def _p1a(src, dst, fsp):
    f = functools.partial(
        pl.kernel,
        out_type=jax.ShapeDtypeStruct((2 * N, 16), jnp.float32),
        mesh=_mesh(),
        scratch_types=(
            [pltpu.VMEM_SHARED((_NPAD, 16), jnp.float32)]
            + [pltpu.VMEM((128,), jnp.int32)] * 6
            + [pltpu.VMEM((128, 8), jnp.float32)] * 6
            + [pltpu.VMEM((128, 16), jnp.float32)]
            + [pltpu.SemaphoreType.DMA] * 12
        ),
        compiler_params=pltpu.CompilerParams(needs_layout_passes=False, use_tc_tiling_on_sc=False),
        name="p1a_geometry",
    )(_p1a_body)
    return f(src, dst, fsp)


def _p1b_body(src_ref, dst_ref, h2_ref, out_ref, *scr):
    acc = scr[0]
    sidx = scr[1:4]
    didx = scr[4:7]
    rows = scr[7:10]
    stage = scr[10]
    si_sem = scr[11:14]
    di_sem = scr[14:17]
    g_sem = scr[17:20]
    c = lax.axis_index("c")
    s = lax.axis_index("s")
    _zero_stage(stage, 16)
    _zero_acc(stage, acc, s)
    plsc.subcore_barrier()

    ub, nu = _tile_range(s, _QB, _RB)
    coff = c * N

    def issue_idx(u, k):
        eb = (ub + u) * 128
        pltpu.async_copy(src_ref.at[pl.ds(eb, 128)], sidx[k], si_sem[k])
        pltpu.async_copy(dst_ref.at[pl.ds(eb, 128)], didx[k], di_sem[k])

    def start_gather(u, k):
        eb = (ub + u) * 128
        pltpu.make_async_copy(src_ref.at[pl.ds(eb, 128)], sidx[k],
                              si_sem[k]).wait()
        pltpu.make_async_copy(dst_ref.at[pl.ds(eb, 128)], didx[k],
                              di_sem[k]).wait()
        sx = sidx[k]
        for gi in range(8):
            sl = pl.ds(gi * 16, 16)
            sx[sl] = sx[sl] + coff
        pltpu.async_copy(h2_ref.at[sx], rows[k], g_sem[k])

    def consume(u, k):
        pltpu.make_async_copy(h2_ref.at[sidx[k]], rows[k], g_sem[k]).wait()
        pltpu.sync_copy(rows[k], acc.at[didx[k]], add=True)

    _sw_pipeline(nu, _QB + 1, issue_idx, start_gather, consume)
    plsc.subcore_barrier()
    _drain_acc(acc, out_ref, s, c * N)


def _p1b(src, dst, h2):
    f = functools.partial(
        pl.kernel,
        out_type=jax.ShapeDtypeStruct((2 * N, 16), jnp.float32),
        mesh=_mesh(),
        scratch_types=(
            [pltpu.VMEM_SHARED((_NPAD, 16), jnp.float32)]
            + [pltpu.VMEM((128,), jnp.int32)] * 6
            + [pltpu.VMEM((128, 16), jnp.float32)] * 3
            + [pltpu.VMEM((128, 16), jnp.float32)]
            + [pltpu.SemaphoreType.DMA] * 9
        ),
        compiler_params=pltpu.CompilerParams(needs_layout_passes=False, use_tc_tiling_on_sc=False),
        name="p1b_agg",
    )(_p1b_body)
    return f(src, dst, h2)


def _p2_body(src_ref, dst_ref, y_ref, out_ref, *scr):
    acc = scr[0]
    sidx = scr[1:4]
    didx = scr[4:7]
    rows = scr[7:10]
    si_sem = scr[10:13]
    di_sem = scr[13:16]
    g_sem = scr[16:19]
    c = lax.axis_index("c")
    s = lax.axis_index("s")
    _zero_stage(rows[0], 8)
    _zero_acc(rows[0], acc, s)
    plsc.subcore_barrier()

    ub, nu = _tile_range(s, _QA, _RA)
    ub = ub + c * _UH

    def issue_idx(u, k):
        eb = (ub + u) * 128
        pltpu.async_copy(src_ref.at[pl.ds(eb, 128)], sidx[k], si_sem[k])
        pltpu.async_copy(dst_ref.at[pl.ds(eb, 128)], didx[k], di_sem[k])

    def start_gather(u, k):
        eb = (ub + u) * 128
        pltpu.make_async_copy(src_ref.at[pl.ds(eb, 128)], sidx[k],
                              si_sem[k]).wait()
        pltpu.make_async_copy(dst_ref.at[pl.ds(eb, 128)], didx[k],
                              di_sem[k]).wait()
        pltpu.async_copy(y_ref.at[sidx[k]], rows[k], g_sem[k])

    def consume(u, k):
        pltpu.make_async_copy(y_ref.at[sidx[k]], rows[k], g_sem[k]).wait()
        pltpu.sync_copy(rows[k], acc.at[didx[k]], add=True)

    _sw_pipeline(nu, _QA + 1, issue_idx, start_gather, consume)
    plsc.subcore_barrier()
    _drain_acc(acc, out_ref, s, c * N)


def _p2(src, dst, y):
    f = functools.partial(
        pl.kernel,
        out_type=jax.ShapeDtypeStruct((2 * N, 8), jnp.float32),
        mesh=_mesh(),
        scratch_types=(
            [pltpu.VMEM_SHARED((_NPAD, 8), jnp.float32)]
            + [pltpu.VMEM((128,), jnp.int32)] * 6
            + [pltpu.VMEM((128, 8), jnp.float32)] * 3
            + [pltpu.SemaphoreType.DMA] * 9
        ),
        compiler_params=pltpu.CompilerParams(needs_layout_passes=False, use_tc_tiling_on_sc=False),
        name="p2_mar",
    )(_p2_body)
    return f(src, dst, y)


def _ka_body(pos_ref, w_ref, out_ref):
    c = pl.program_id(0)
    h = jnp.maximum(
        jnp.dot(pos_ref[...], w_ref[...], preferred_element_type=jnp.float32),
        0.0)
    out_ref[...] = jnp.where(c == 0, h[:, :16], h[:, 16:])


def _kernel_a(pos, W_fe1):
    return pl.pallas_call(
        _ka_body,
        grid=(NC, NB),
        in_specs=[
            pl.BlockSpec((BN, 2), lambda c, i: (i, 0)),
            pl.BlockSpec((2, N_FE), lambda c, i: (0, 0)),
        ],
        out_specs=pl.BlockSpec((BN, 16), lambda c, i: (c * NB + i, 0)),
        out_shape=jax.ShapeDtypeStruct((2 * N, 16), jnp.float32),
    )(pos, W_fe1)


def _kb_body(h1_ref, h2_ref, g1_ref, g2_ref, a1_ref, a2_ref,
             wfe2_ref, wg0_ref, wb0_ref, wout0_ref, wg1_ref, wb1_ref, wout1_ref,
             y_ref):
    h = jnp.concatenate([h1_ref[...], h2_ref[...]], axis=1)
    geo = g1_ref[...] + g2_ref[...]
    deg = jnp.maximum(geo[:, 0:1], 1.0)
    agg = jnp.concatenate([a1_ref[...], a2_ref[...]], axis=1) / deg
    wfe2 = wfe2_ref[...]
    learned = jnp.maximum(
        jnp.dot(h, wfe2[:N_FE], preferred_element_type=jnp.float32)
        + jnp.dot(agg, wfe2[N_FE:], preferred_element_type=jnp.float32), 0.0)

    lap0 = geo[:, 1:2] / deg
    lap1 = geo[:, 2:3] / deg
    lap2 = geo[:, 3:4] / deg
    a00 = geo[:, 4:5] + 1e-4
    a01 = geo[:, 5:6]
    a11 = geo[:, 6:7] + 1e-4
    b00 = geo[:, 7:8]
    b01 = geo[:, 8:9]
    b10 = geo[:, 9:10]
    b11 = geo[:, 10:11]
    det = a00 * a11 - a01 * a01
    g00 = (a11 * b00 - a01 * b10) / det
    g01 = (a11 * b01 - a01 * b11) / det
    g10 = (a00 * b10 - a01 * b00) / det
    g11 = (a00 * b11 - a01 * b01) / det
    exx = g00
    eyy = g11
    exy = 0.5 * (g01 + g10)
    vm = jnp.sqrt(exx * exx - exx * eyy + eyy * eyy + 3.0 * exy * exy + 1e-12)
    vol = exx + eyy

    mod0 = learned * (1.0 + lap0 * wg0_ref[...]) + lap0 * wb0_ref[...]
    y0 = jnp.dot(mod0, wout0_ref[...], preferred_element_type=jnp.float32)
    mask1 = jnp.concatenate([exx, eyy, exy, vm, vol, lap1, lap2], axis=1)
    mod1 = (learned * (1.0 + jnp.dot(mask1, wg1_ref[...],
                                     preferred_element_type=jnp.float32))
            + jnp.dot(mask1, wb1_ref[...], preferred_element_type=jnp.float32))
    y12 = jnp.dot(mod1, wout1_ref[...], preferred_element_type=jnp.float32)
    y_ref[...] = jnp.concatenate(
        [y0, y12, jnp.zeros((y0.shape[0], 5), jnp.float32)], axis=1)


def _kernel_b(h2, geo, agg, W_fe2, Wg0, Wb0, Wout0, Wg1, Wb1, Wout1):
    full = lambda shape: pl.BlockSpec(shape, lambda i: (0, 0))
    return pl.pallas_call(
        _kb_body,
        grid=(NB,),
        in_specs=[
            pl.BlockSpec((BN, 16), lambda i: (i, 0)),
            pl.BlockSpec((BN, 16), lambda i: (NB + i, 0)),
            pl.BlockSpec((BN, 16), lambda i: (i, 0)),
            pl.BlockSpec((BN, 16), lambda i: (NB + i, 0)),
            pl.BlockSpec((BN, 16), lambda i: (i, 0)),
            pl.BlockSpec((BN, 16), lambda i: (NB + i, 0)),
            full((2 * N_FE, N_FE)),
            full((1, N_FE)),
            full((1, N_FE)),
            full((N_FE, 1)),
            full((7, N_FE)),
            full((7, N_FE)),
            full((N_FE, 2)),
        ],
        out_specs=pl.BlockSpec((BN, 8), lambda i: (i, 0)),
        out_shape=jax.ShapeDtypeStruct((N, 8), jnp.float32),
    )(h2, h2, geo, geo, agg, agg, W_fe2, Wg0, Wb0, Wout0, Wg1, Wb1, Wout1)


def _kc_body(ya_ref, yb_ref, g1_ref, g2_ref, out_ref):
    deg = jnp.maximum(g1_ref[:, 0:1] + g2_ref[:, 0:1], 1.0)
    out_ref[...] = (ya_ref[:, :3] + yb_ref[:, :3]) / deg


def _kernel_c(acc_y, geo):
    return pl.pallas_call(
        _kc_body,
        grid=(NB,),
        in_specs=[
            pl.BlockSpec((BN, 8), lambda i: (i, 0)),
            pl.BlockSpec((BN, 8), lambda i: (NB + i, 0)),
            pl.BlockSpec((BN, 16), lambda i: (i, 0)),
            pl.BlockSpec((BN, 16), lambda i: (NB + i, 0)),
        ],
        out_specs=pl.BlockSpec((BN, 3), lambda i: (i, 0)),
        out_shape=jax.ShapeDtypeStruct((N, 3), jnp.float32),
    )(acc_y, acc_y, geo, geo)


def kernel(full_state, edge_index, W_fe1, W_fe2, Wg0, Wb0, Wout0, Wg1, Wb1, Wout1):
    src = edge_index[0]
    dst = edge_index[1]
    fsp = jnp.concatenate(
        [full_state, jnp.zeros((N, 3), jnp.float32)], axis=1)
    h2 = _kernel_a(full_state[:, :2], W_fe1)
    geo = _p1a(src, dst, fsp)
    agg = _p1b(src, dst, h2)
    y = _kernel_b(h2, geo, agg, W_fe2, Wg0, Wb0, Wout0, Wg1, Wb1, Wout1)
    acc_y = _p2(src, dst, y)
    return _kernel_c(acc_y, geo)
